# baseline (speedup 1.0000x reference)
import functools

import jax
import jax.numpy as jnp
from jax.experimental import pallas as pl
from jax.experimental.pallas import tpu as pltpu

B = 16384
M_DIM = 9
C = 32


def _tp_body(w_ref, x1_ref, x2_ref, out_ref):
    x1 = x1_ref[...]
    x2 = x2_ref[...]
    acc = [None] * M_DIM
    for m1 in range(M_DIM):
        a = x1[:, m1, :]
        for m2 in range(M_DIM):
            p = a * x2[:, m2, :]
            for m in range(M_DIM):
                term = w_ref[m, m1, m2] * p
                acc[m] = term if acc[m] is None else acc[m] + term
    for m in range(M_DIM):
        out_ref[:, m, :] = acc[m]


def kernel(x1, x2, CG_vals, M1, M2, M_ptr):
    seg_lens = M_ptr[1:] - M_ptr[:-1]
    nnz = M1.shape[0]
    seg_ids = jnp.repeat(
        jnp.arange(M_DIM, dtype=jnp.int32), seg_lens, total_repeat_length=nnz
    )
    w = jnp.zeros((M_DIM, M_DIM, M_DIM), jnp.float32)
    w = w.at[seg_ids, M1, M2].add(CG_vals)

    bB = 256
    grid = (B // bB,)
    out = pl.pallas_call(
        _tp_body,
        grid=grid,
        in_specs=[
            pl.BlockSpec(memory_space=pltpu.SMEM),
            pl.BlockSpec((bB, M_DIM, C), lambda i: (i, 0, 0)),
            pl.BlockSpec((bB, M_DIM, C), lambda i: (i, 0, 0)),
        ],
        out_specs=pl.BlockSpec((bB, M_DIM, C), lambda i: (i, 0, 0)),
        out_shape=jax.ShapeDtypeStruct((B, M_DIM, C), jnp.float32),
    )(w, x1, x2)
    return out


# speedup vs baseline: 5.8524x; 5.8524x over previous
import functools

import jax
import jax.numpy as jnp
from jax import lax
from jax.experimental import pallas as pl
from jax.experimental.pallas import tpu as pltpu
from jax.experimental.pallas import tpu_sc as plsc

B = 16384
M_DIM = 9
C = 32
NNZ = 90
ROW = M_DIM * C
NC, NS, L = 2, 16, 16
NW = NC * NS
RW = B // NW
R = 64
NCHUNK = RW // R
NJ = NNZ * C // L
CW = R * ROW


def _sc_body(x1_hbm, x2_hbm, i1_hbm, i2_hbm, io_hbm, cg_hbm, out_hbm,
             x1c, x2c, outc, i1v, i2v, iov, cgv):
    wid = lax.axis_index("s") * NC + lax.axis_index("c")
    base = wid * (RW * ROW)
    pltpu.sync_copy(i1_hbm, i1v)
    pltpu.sync_copy(i2_hbm, i2v)
    pltpu.sync_copy(io_hbm, iov)
    pltpu.sync_copy(cg_hbm, cgv)

    def chunk_body(ci, carry):
        off = base + ci * CW
        pltpu.sync_copy(x1_hbm.at[pl.ds(off, CW)], x1c)
        pltpu.sync_copy(x2_hbm.at[pl.ds(off, CW)], x2c)

        zero = jnp.zeros((L,), jnp.float32)

        def zbody(q, c):
            for k in range(8):
                outc[pl.ds((q * 8 + k) * L, L)] = zero
            return c
        lax.fori_loop(0, CW // L // 8, zbody, 0)

        def jbody(j, c):
            i1 = i1v[pl.ds(j * L, L)]
            i2 = i2v[pl.ds(j * L, L)]
            io = iov[pl.ds(j * L, L)]
            cg = cgv[pl.ds(j * L, L)]

            def rbody(r, cc):
                rs = jnp.full((L,), r * ROW, jnp.int32)
                a = plsc.load_gather(x1c, [i1 + rs])
                b = plsc.load_gather(x2c, [i2 + rs])
                plsc.addupdate_scatter(outc, [io + rs], a * b * cg)
                return cc
            lax.fori_loop(0, R, rbody, 0)
            return c
        lax.fori_loop(0, NJ, jbody, 0)

        pltpu.sync_copy(outc, out_hbm.at[pl.ds(off, CW)])
        return carry
    lax.fori_loop(0, NCHUNK, chunk_body, 0)


def kernel(x1, x2, CG_vals, M1, M2, M_ptr):
    seg_lens = M_ptr[1:] - M_ptr[:-1]
    seg_ids = jnp.repeat(
        jnp.arange(M_DIM, dtype=jnp.int32), seg_lens, total_repeat_length=NNZ
    )
    lanes = jnp.arange(C, dtype=jnp.int32)[None, :]
    i1 = (M1[:, None] * C + lanes).reshape(NJ * L)
    i2 = (M2[:, None] * C + lanes).reshape(NJ * L)
    io = (seg_ids[:, None] * C + lanes).reshape(NJ * L)
    cg = jnp.broadcast_to(CG_vals[:, None], (NNZ, C)).reshape(NJ * L)

    x1f = x1.reshape(B * ROW)
    x2f = x2.reshape(B * ROW)

    mesh = plsc.VectorSubcoreMesh(
        core_axis_name="c", subcore_axis_name="s", num_cores=NC, num_subcores=NS
    )
    out = pl.kernel(
        _sc_body,
        out_type=jax.ShapeDtypeStruct((B * ROW,), jnp.float32),
        mesh=mesh,
        compiler_params=pltpu.CompilerParams(needs_layout_passes=False),
        scratch_types=[
            pltpu.VMEM((CW,), jnp.float32),
            pltpu.VMEM((CW,), jnp.float32),
            pltpu.VMEM((CW,), jnp.float32),
            pltpu.VMEM((NJ * L,), jnp.int32),
            pltpu.VMEM((NJ * L,), jnp.int32),
            pltpu.VMEM((NJ * L,), jnp.int32),
            pltpu.VMEM((NJ * L,), jnp.float32),
        ],
    )(x1f, x2f, i1, i2, io, cg)
    return out.reshape(B, M_DIM, C)


# speedup vs baseline: 11.2756x; 1.9267x over previous
import functools

import jax
import jax.numpy as jnp
from jax import lax
from jax.experimental import pallas as pl
from jax.experimental.pallas import tpu as pltpu
from jax.experimental.pallas import tpu_sc as plsc

B = 16384
M_DIM = 9
C = 32
NNZ = 90
ROW = M_DIM * C
NC, NS, L = 2, 16, 16
NW = NC * NS
RW = B // NW
R = 64
NCHUNK = RW // R
NJ = NNZ * C // L
CW = R * ROW


def _sc_body(x1_hbm, x2_hbm, i1_hbm, i2_hbm, io_hbm, cg_hbm, out_hbm,
             x1c, x2c, outc, i1v, i2v, iov, cgv):
    wid = lax.axis_index("s") * NC + lax.axis_index("c")
    base = wid * (RW * ROW)
    pltpu.sync_copy(i1_hbm, i1v)
    pltpu.sync_copy(i2_hbm, i2v)
    pltpu.sync_copy(io_hbm, iov)
    pltpu.sync_copy(cg_hbm, cgv)

    def chunk_body(ci, carry):
        off = base + ci * CW
        pltpu.sync_copy(x1_hbm.at[pl.ds(off, CW)], x1c)
        pltpu.sync_copy(x2_hbm.at[pl.ds(off, CW)], x2c)

        zero = jnp.zeros((L,), jnp.float32)

        @plsc.parallel_loop(0, CW // L, step=1, unroll=8)
        def zbody(q):
            outc[pl.ds(q * L, L)] = zero

        def jbody(j, c):
            i1 = i1v[pl.ds(j * L, L)]
            i2 = i2v[pl.ds(j * L, L)]
            io = iov[pl.ds(j * L, L)]
            cg = cgv[pl.ds(j * L, L)]

            @plsc.parallel_loop(0, R, step=1, unroll=8)
            def rbody(r):
                rs = jnp.full((L,), r * ROW, jnp.int32)
                a = plsc.load_gather(x1c, [i1 + rs])
                b = plsc.load_gather(x2c, [i2 + rs])
                plsc.addupdate_scatter(outc, [io + rs], a * b * cg)
            return c
        lax.fori_loop(0, NJ, jbody, 0)

        pltpu.sync_copy(outc, out_hbm.at[pl.ds(off, CW)])
        return carry
    lax.fori_loop(0, NCHUNK, chunk_body, 0)


def kernel(x1, x2, CG_vals, M1, M2, M_ptr):
    seg_lens = M_ptr[1:] - M_ptr[:-1]
    seg_ids = jnp.repeat(
        jnp.arange(M_DIM, dtype=jnp.int32), seg_lens, total_repeat_length=NNZ
    )
    lanes = jnp.arange(C, dtype=jnp.int32)[None, :]
    i1 = (M1[:, None] * C + lanes).reshape(NJ * L)
    i2 = (M2[:, None] * C + lanes).reshape(NJ * L)
    io = (seg_ids[:, None] * C + lanes).reshape(NJ * L)
    cg = jnp.broadcast_to(CG_vals[:, None], (NNZ, C)).reshape(NJ * L)

    x1f = x1.reshape(B * ROW)
    x2f = x2.reshape(B * ROW)

    mesh = plsc.VectorSubcoreMesh(
        core_axis_name="c", subcore_axis_name="s", num_cores=NC, num_subcores=NS
    )
    out = pl.kernel(
        _sc_body,
        out_type=jax.ShapeDtypeStruct((B * ROW,), jnp.float32),
        mesh=mesh,
        compiler_params=pltpu.CompilerParams(needs_layout_passes=False),
        scratch_types=[
            pltpu.VMEM((CW,), jnp.float32),
            pltpu.VMEM((CW,), jnp.float32),
            pltpu.VMEM((CW,), jnp.float32),
            pltpu.VMEM((NJ * L,), jnp.int32),
            pltpu.VMEM((NJ * L,), jnp.int32),
            pltpu.VMEM((NJ * L,), jnp.int32),
            pltpu.VMEM((NJ * L,), jnp.float32),
        ],
    )(x1f, x2f, i1, i2, io, cg)
    return out.reshape(B, M_DIM, C)


# speedup vs baseline: 11.4071x; 1.0117x over previous
import functools

import jax
import jax.numpy as jnp
from jax import lax
from jax.experimental import pallas as pl
from jax.experimental.pallas import tpu as pltpu
from jax.experimental.pallas import tpu_sc as plsc

B = 16384
M_DIM = 9
C = 32
NNZ = 90
ROW = M_DIM * C
NC, NS, L = 2, 16, 16
NW = NC * NS
RW = B // NW
R = 64
NCHUNK = RW // R
NJ = NNZ * C // L
CW = R * ROW


def _sc_body(x1_hbm, x2_hbm, i1_hbm, i2_hbm, io_hbm, cg_hbm, out_hbm,
             x1c, x2c, outc, i1v, i2v, iov, cgv):
    wid = lax.axis_index("s") * NC + lax.axis_index("c")
    base = wid * (RW * ROW)
    pltpu.sync_copy(i1_hbm, i1v)
    pltpu.sync_copy(i2_hbm, i2v)
    pltpu.sync_copy(io_hbm, iov)
    pltpu.sync_copy(cg_hbm, cgv)

    def chunk_body(ci, carry):
        off = base + ci * CW
        pltpu.sync_copy(x1_hbm.at[pl.ds(off, CW)], x1c)
        pltpu.sync_copy(x2_hbm.at[pl.ds(off, CW)], x2c)

        zero = jnp.zeros((L,), jnp.float32)

        @plsc.parallel_loop(0, CW // L, step=1, unroll=8)
        def zbody(q):
            outc[pl.ds(q * L, L)] = zero

        def jbody(j, c):
            i1 = i1v[pl.ds(j * L, L)]
            i2 = i2v[pl.ds(j * L, L)]
            io = iov[pl.ds(j * L, L)]
            cg = cgv[pl.ds(j * L, L)]

            @plsc.parallel_loop(0, CW, step=ROW, unroll=8)
            def rbody(r):
                a = plsc.load_gather(x1c.at[pl.ds(r, ROW)], [i1])
                b = plsc.load_gather(x2c.at[pl.ds(r, ROW)], [i2])
                plsc.addupdate_scatter(outc.at[pl.ds(r, ROW)], [io], a * b * cg)
            return c
        lax.fori_loop(0, NJ, jbody, 0)

        pltpu.sync_copy(outc, out_hbm.at[pl.ds(off, CW)])
        return carry
    lax.fori_loop(0, NCHUNK, chunk_body, 0)


def kernel(x1, x2, CG_vals, M1, M2, M_ptr):
    seg_lens = M_ptr[1:] - M_ptr[:-1]
    seg_ids = jnp.repeat(
        jnp.arange(M_DIM, dtype=jnp.int32), seg_lens, total_repeat_length=NNZ
    )
    lanes = jnp.arange(C, dtype=jnp.int32)[None, :]
    i1 = (M1[:, None] * C + lanes).reshape(NJ * L)
    i2 = (M2[:, None] * C + lanes).reshape(NJ * L)
    io = (seg_ids[:, None] * C + lanes).reshape(NJ * L)
    cg = jnp.broadcast_to(CG_vals[:, None], (NNZ, C)).reshape(NJ * L)

    x1f = x1.reshape(B * ROW)
    x2f = x2.reshape(B * ROW)

    mesh = plsc.VectorSubcoreMesh(
        core_axis_name="c", subcore_axis_name="s", num_cores=NC, num_subcores=NS
    )
    out = pl.kernel(
        _sc_body,
        out_type=jax.ShapeDtypeStruct((B * ROW,), jnp.float32),
        mesh=mesh,
        compiler_params=pltpu.CompilerParams(needs_layout_passes=False),
        scratch_types=[
            pltpu.VMEM((CW,), jnp.float32),
            pltpu.VMEM((CW,), jnp.float32),
            pltpu.VMEM((CW,), jnp.float32),
            pltpu.VMEM((NJ * L,), jnp.int32),
            pltpu.VMEM((NJ * L,), jnp.int32),
            pltpu.VMEM((NJ * L,), jnp.int32),
            pltpu.VMEM((NJ * L,), jnp.float32),
        ],
    )(x1f, x2f, i1, i2, io, cg)
    return out.reshape(B, M_DIM, C)
